# baseline (speedup 1.0000x reference)
import functools

import jax
import jax.numpy as jnp
from jax import lax
from jax.experimental import pallas as pl
from jax.experimental.pallas import tpu as pltpu
from jax.experimental.pallas import tpu_sc as plsc

_K = 64


def _build_sc_kernel(d, n_cfg, mid, n_half, n_acc, rows_main, rows_last,
                     zrows, zlast, e_pad):
  mesh = plsc.VectorSubcoreMesh(core_axis_name="c", subcore_axis_name="s")

  @functools.partial(
      pl.kernel,
      mesh=mesh,
      compiler_params=pltpu.CompilerParams(use_tc_tiling_on_sc=True),
      out_type=jax.ShapeDtypeStruct((n_cfg, d // 128, 128), jnp.float32),
      scratch_types=[
          pltpu.VMEM((32,), jnp.int32),
          pltpu.VMEM((_K,), jnp.int32),
          pltpu.VMEM((_K,), jnp.int32),
          pltpu.VMEM((_K,), jnp.int32),
          pltpu.VMEM((_K,), jnp.int32),
          pltpu.VMEM((_K,), jnp.int32),
          pltpu.VMEM((_K, d // 128, 128), jnp.float32),
          pltpu.VMEM((_K, d // 128, 128), jnp.float32),
          pltpu.VMEM_SHARED((n_acc, d // 128, 128), jnp.float32),
          pltpu.SemaphoreType.DMA,
          pltpu.SemaphoreType.DMA,
          pltpu.SemaphoreType.DMA,
      ],
  )
  def body(table_hbm, keys_hbm, segs_hbm, params_hbm, zeros_hbm, out_hbm,
           prm, key0, key1, segr, loc0, loc1, rows0, rows1, acc,
           sem0, sem1, isem):
    c = lax.axis_index("c")
    s = lax.axis_index("s")

    pltpu.sync_copy(params_hbm, prm)

    @pl.when(s < 15)
    def _():
      pltpu.sync_copy(zeros_hbm.at[pl.ds(0, zrows)],
                      acc.at[pl.ds(s * zrows, zrows)])

    @pl.when(s == 15)
    def _():
      pltpu.sync_copy(zeros_hbm.at[pl.ds(0, zlast)],
                      acc.at[pl.ds(15 * zrows, zlast)])

    pv = prm[pl.ds(pl.multiple_of(c * 8, 8), 16)]
    n_chunks = pv[0]
    a_c = pv[1]
    base_seg = c * mid
    n_my = jnp.maximum(0, (n_chunks - s + 15) // 16)
    plsc.subcore_barrier()

    def prep(t, keyb, locb, rowsb, sem):
      base = pl.multiple_of(a_c, 8) + (t * 16 + s) * _K
      pltpu.async_copy(keys_hbm.at[pl.ds(base, _K)], keyb, isem)
      pltpu.async_copy(segs_hbm.at[pl.ds(base, _K)], segr, isem)
      pltpu.make_async_copy(keys_hbm.at[pl.ds(base, _K)], keyb, isem).wait()
      pltpu.make_async_copy(segs_hbm.at[pl.ds(base, _K)], segr, isem).wait()
      for i in range(_K // 16):
        sl = pl.ds(i * 16, 16)
        sv = segr[sl] - base_seg
        ok = (sv >= 0) & (sv < mid)
        locb[sl] = jnp.where(ok, sv, mid)
      return pltpu.async_copy(table_hbm.at[keyb], rowsb, sem)

    @pl.when(n_my > 0)
    def _():
      prep(0, key0, loc0, rows0, sem0)

    def step(t, carry):
      a = 2 * t
      b = 2 * t + 1

      @pl.when(b < n_my)
      def _():
        prep(b, key1, loc1, rows1, sem1)

      pltpu.make_async_copy(table_hbm.at[key0], rows0, sem0).wait()
      pltpu.sync_copy(rows0, acc.at[loc0], add=True)

      @pl.when(b + 1 < n_my)
      def _():
        prep(b + 1, key0, loc0, rows0, sem0)

      @pl.when(b < n_my)
      def _():
        pltpu.make_async_copy(table_hbm.at[key1], rows1, sem1).wait()
        pltpu.sync_copy(rows1, acc.at[loc1], add=True)

      return carry

    lax.fori_loop(0, (n_my + 1) // 2, step, 0)
    plsc.subcore_barrier()

    @pl.when(s < 15)
    def _():
      r0 = pl.multiple_of(s * rows_main, 8)
      pltpu.sync_copy(acc.at[pl.ds(r0, rows_main)],
                      out_hbm.at[pl.ds(c * mid + r0, rows_main)])

    @pl.when(s == 15)
    def _():
      r0 = 15 * rows_main
      pltpu.sync_copy(acc.at[pl.ds(r0, rows_last)],
                      out_hbm.at[pl.ds(c * mid + r0, rows_last)])

  return body


def kernel(ast_nodes_encodings,
           ast_node_idx_to_pdg_node_idx_mapping_key,
           ast_node_idx_to_pdg_node_idx_mapping_value,
           pdg_node_idx_to_sub_ast_root_idx_mapping_key,
           pdg_node_idx_to_sub_ast_root_idx_mapping_value,
           nr_cfg_nodes):
  table = ast_nodes_encodings
  keys = ast_node_idx_to_pdg_node_idx_mapping_key
  segs = ast_node_idx_to_pdg_node_idx_mapping_value
  n_ast, d = table.shape
  e = keys.shape[0]
  n_cfg = pdg_node_idx_to_sub_ast_root_idx_mapping_key.shape[0]
  mid = n_cfg // 2

  e_pad = -(-e // _K) * _K
  pad = e_pad - e
  keys_p = jnp.concatenate(
      [keys.astype(jnp.int32), jnp.zeros((pad,), jnp.int32)])
  segs_p = jnp.concatenate(
      [segs.astype(jnp.int32), jnp.full((pad,), n_cfg, jnp.int32)])

  p = jnp.searchsorted(segs_p, mid).astype(jnp.int32)
  a1 = (p // _K) * _K
  count0 = (p + _K - 1) // _K
  count1 = (e_pad - a1) // _K
  params = jnp.zeros((32,), jnp.int32)
  params = params.at[0].set(count0).at[8].set(count1).at[9].set(a1)

  n_half = n_cfg - mid
  n_acc = -(-(mid + 1) // 8) * 8
  zrows = -(-n_acc // (16 * 8)) * 8
  zlast = n_acc - 15 * zrows
  rows_main = (mid // (16 * 8)) * 8
  rows_last = mid - 15 * rows_main
  zeros = jnp.zeros((max(zrows, zlast), d // 128, 128), jnp.float32)

  body = _build_sc_kernel(d, n_cfg, mid, n_half, n_acc, rows_main,
                          rows_last, zrows, zlast, e_pad)
  out = body(table.reshape(n_ast, d // 128, 128), keys_p, segs_p, params,
             zeros)
  return out.reshape(n_cfg, d)


# speedup vs baseline: 1.1189x; 1.1189x over previous
import functools

import jax
import jax.numpy as jnp
from jax import lax
from jax.experimental import pallas as pl
from jax.experimental.pallas import tpu as pltpu
from jax.experimental.pallas import tpu_sc as plsc

_K = 96


def _build_sc_kernel(d, n_cfg, mid, n_half, n_acc, rows_main, rows_last,
                     zrows, zlast, e_pad):
  mesh = plsc.VectorSubcoreMesh(core_axis_name="c", subcore_axis_name="s")

  @functools.partial(
      pl.kernel,
      mesh=mesh,
      compiler_params=pltpu.CompilerParams(use_tc_tiling_on_sc=True),
      out_type=jax.ShapeDtypeStruct((n_cfg, d // 128, 128), jnp.float32),
      scratch_types=[
          pltpu.VMEM((32,), jnp.int32),
          pltpu.VMEM((_K,), jnp.int32),
          pltpu.VMEM((_K,), jnp.int32),
          pltpu.VMEM((_K,), jnp.int32),
          pltpu.VMEM((_K,), jnp.int32),
          pltpu.VMEM((_K,), jnp.int32),
          pltpu.VMEM((_K, d // 128, 128), jnp.float32),
          pltpu.VMEM((_K, d // 128, 128), jnp.float32),
          pltpu.VMEM_SHARED((n_acc, d // 128, 128), jnp.float32),
          pltpu.SemaphoreType.DMA,
          pltpu.SemaphoreType.DMA,
          pltpu.SemaphoreType.DMA,
      ],
  )
  def body(table_hbm, keys_hbm, segs_hbm, params_hbm, zeros_hbm, out_hbm,
           prm, key0, key1, segr, loc0, loc1, rows0, rows1, acc,
           sem0, sem1, isem):
    c = lax.axis_index("c")
    s = lax.axis_index("s")

    pltpu.sync_copy(params_hbm, prm)

    @pl.when(s < 15)
    def _():
      pltpu.sync_copy(zeros_hbm.at[pl.ds(0, zrows)],
                      acc.at[pl.ds(s * zrows, zrows)])

    @pl.when(s == 15)
    def _():
      pltpu.sync_copy(zeros_hbm.at[pl.ds(0, zlast)],
                      acc.at[pl.ds(15 * zrows, zlast)])

    pv = prm[pl.ds(pl.multiple_of(c * 8, 8), 16)]
    n_chunks = pv[0]
    a_c = pv[1]
    base_seg = c * mid
    n_my = jnp.maximum(0, (n_chunks - s + 15) // 16)
    plsc.subcore_barrier()

    def prep(t, keyb, locb, rowsb, sem):
      base = pl.multiple_of(a_c, 8) + (t * 16 + s) * _K
      pltpu.async_copy(keys_hbm.at[pl.ds(base, _K)], keyb, isem)
      pltpu.async_copy(segs_hbm.at[pl.ds(base, _K)], segr, isem)
      pltpu.make_async_copy(keys_hbm.at[pl.ds(base, _K)], keyb, isem).wait()
      pltpu.make_async_copy(segs_hbm.at[pl.ds(base, _K)], segr, isem).wait()
      for i in range(_K // 16):
        sl = pl.ds(i * 16, 16)
        sv = segr[sl] - base_seg
        ok = (sv >= 0) & (sv < mid)
        locb[sl] = jnp.where(ok, sv, mid)
      return pltpu.async_copy(table_hbm.at[keyb], rowsb, sem)

    @pl.when(n_my > 0)
    def _():
      prep(0, key0, loc0, rows0, sem0)

    def step(t, carry):
      a = 2 * t
      b = 2 * t + 1

      @pl.when(b < n_my)
      def _():
        prep(b, key1, loc1, rows1, sem1)

      pltpu.make_async_copy(table_hbm.at[key0], rows0, sem0).wait()
      pltpu.sync_copy(rows0, acc.at[loc0], add=True)

      @pl.when(b + 1 < n_my)
      def _():
        prep(b + 1, key0, loc0, rows0, sem0)

      @pl.when(b < n_my)
      def _():
        pltpu.make_async_copy(table_hbm.at[key1], rows1, sem1).wait()
        pltpu.sync_copy(rows1, acc.at[loc1], add=True)

      return carry

    lax.fori_loop(0, (n_my + 1) // 2, step, 0)
    plsc.subcore_barrier()

    @pl.when(s < 15)
    def _():
      r0 = pl.multiple_of(s * rows_main, 8)
      pltpu.sync_copy(acc.at[pl.ds(r0, rows_main)],
                      out_hbm.at[pl.ds(c * mid + r0, rows_main)])

    @pl.when(s == 15)
    def _():
      r0 = 15 * rows_main
      pltpu.sync_copy(acc.at[pl.ds(r0, rows_last)],
                      out_hbm.at[pl.ds(c * mid + r0, rows_last)])

  return body


def kernel(ast_nodes_encodings,
           ast_node_idx_to_pdg_node_idx_mapping_key,
           ast_node_idx_to_pdg_node_idx_mapping_value,
           pdg_node_idx_to_sub_ast_root_idx_mapping_key,
           pdg_node_idx_to_sub_ast_root_idx_mapping_value,
           nr_cfg_nodes):
  table = ast_nodes_encodings
  keys = ast_node_idx_to_pdg_node_idx_mapping_key
  segs = ast_node_idx_to_pdg_node_idx_mapping_value
  n_ast, d = table.shape
  e = keys.shape[0]
  n_cfg = pdg_node_idx_to_sub_ast_root_idx_mapping_key.shape[0]
  mid = n_cfg // 2

  e_pad = -(-e // _K) * _K
  pad = e_pad - e
  keys_p = jnp.concatenate(
      [keys.astype(jnp.int32), jnp.zeros((pad,), jnp.int32)])
  segs_p = jnp.concatenate(
      [segs.astype(jnp.int32), jnp.full((pad,), n_cfg, jnp.int32)])

  p = jnp.sum(segs_p < mid).astype(jnp.int32)
  a1 = (p // _K) * _K
  count0 = (p + _K - 1) // _K
  count1 = (e_pad - a1) // _K
  params = jnp.zeros((32,), jnp.int32)
  params = params.at[0].set(count0).at[8].set(count1).at[9].set(a1)

  n_half = n_cfg - mid
  n_acc = -(-(mid + 1) // 8) * 8
  zrows = -(-n_acc // (16 * 8)) * 8
  zlast = n_acc - 15 * zrows
  rows_main = (mid // (16 * 8)) * 8
  rows_last = mid - 15 * rows_main
  zeros = jnp.zeros((max(zrows, zlast), d // 128, 128), jnp.float32)

  body = _build_sc_kernel(d, n_cfg, mid, n_half, n_acc, rows_main,
                          rows_last, zrows, zlast, e_pad)
  out = body(table.reshape(n_ast, d // 128, 128), keys_p, segs_p, params,
             zeros)
  return out.reshape(n_cfg, d)
